# Initial kernel scaffold; baseline (speedup 1.0000x reference)
import functools

import jax
import jax.numpy as jnp
from jax import lax
from jax.experimental import pallas as pl
from jax.experimental.pallas import tpu as pltpu
from jax.experimental.pallas import tpu_sc as plsc

_NC = 2
_NS = 16
_NW = _NC * _NS
_CR = 128


@functools.lru_cache(maxsize=None)
def _build(B, N, D, M, K):
    WPB = _NW // B
    MW = M // WPB
    ROWS_W = MW * K
    NCH = ROWS_W // _CR
    ICH = NCH
    MPC = _CR // K

    mesh = plsc.VectorSubcoreMesh(core_axis_name="c", subcore_axis_name="s")

    def body(feat_hbm, pts_hbm, next_hbm, idx_hbm, fout_hbm, pout_hbm,
             idx_ref, ptsv, nextv, fb0, fb1, po, gs0, gs1, ws0, ws1):
        wid = lax.axis_index("s") * _NC + lax.axis_index("c")
        b = wid // WPB
        q = wid % WPB

        pltpu.sync_copy(idx_hbm.at[b, pl.ds(q * ICH, ICH)], idx_ref)
        pltpu.sync_copy(pts_hbm.at[b], ptsv)
        pltpu.sync_copy(next_hbm.at[b, pl.ds(q * MW, MW)], nextv)

        fbufs = (fb0, fb1)
        gsems = (gs0, gs1)
        wsems = (ws0, ws1)
        feat_b = feat_hbm.at[b]
        row0 = q * ROWS_W

        def fire_gather(c, j2):
            pltpu.make_async_copy(
                feat_b.at[idx_ref.at[c]], fbufs[j2], gsems[j2]).start()

        def wait_gather(j2):
            pltpu.make_async_copy(
                feat_b.at[idx_ref.at[0]], fbufs[j2], gsems[j2]).wait()

        def fire_write(c, j2):
            pltpu.make_async_copy(
                fbufs[j2], fout_hbm.at[b, pl.ds(row0 + c * _CR, _CR)],
                wsems[j2]).start()

        def wait_write(j2):
            pltpu.make_async_copy(
                fbufs[j2], fout_hbm.at[b, pl.ds(row0, _CR)], wsems[j2]).wait()

        z16 = jnp.zeros((K,), jnp.int32)
        o16 = z16 + 1
        t16 = z16 + 2
        ki3 = lax.iota(jnp.int32, K) * 3
        zf16 = jnp.zeros((K,), jnp.float32)

        def pts_chunk(c):
            for j in range(MPC):
                m = c * MPC + j
                m16 = z16 + m
                iv = idx_ref[c, pl.ds(j * K, K)]
                px = plsc.load_gather(ptsv, [iv, z16])
                py = plsc.load_gather(ptsv, [iv, o16])
                pz = plsc.load_gather(ptsv, [iv, t16])
                dx = px - plsc.load_gather(nextv, [m16, z16])
                dy = py - plsc.load_gather(nextv, [m16, o16])
                dz = pz - plsc.load_gather(nextv, [m16, t16])
                d2 = dx * dx + dy * dy + dz * dz
                mv = zf16 + jnp.max(d2)
                bits = plsc.bitcast(mv, jnp.int32)
                y = plsc.bitcast(jnp.int32(0x5F3759DF) - (bits >> 1),
                                 jnp.float32)
                half = mv * 0.5
                for _ in range(3):
                    y = y * (1.5 - half * y * y)
                scale = jnp.where(mv == 0.0, jnp.float32(1.0), y)
                plsc.store_scatter(po, [m16, ki3], dx * scale)
                plsc.store_scatter(po, [m16, ki3 + 1], dy * scale)
                plsc.store_scatter(po, [m16, ki3 + 2], dz * scale)

        fire_gather(0, 0)
        fire_gather(1, 1)

        def pair(cc, carry):
            for j2 in range(2):
                c = cc * 2 + j2
                wait_gather(j2)
                fire_write(c, j2)
                pts_chunk(c)
                wait_write(j2)
                fire_gather(c + 2, j2)
            return carry

        lax.fori_loop(0, NCH // 2 - 1, pair, 0)
        for j2 in range(2):
            c = NCH - 2 + j2
            wait_gather(j2)
            fire_write(c, j2)
            pts_chunk(c)
            wait_write(j2)

        pltpu.sync_copy(po, pout_hbm.at[b, pl.ds(q * MW, MW)])

    return pl.kernel(
        body,
        out_type=[
            jax.ShapeDtypeStruct((B, M * K, D), jnp.float32),
            jax.ShapeDtypeStruct((B, M, 3 * K), jnp.float32),
        ],
        mesh=mesh,
        scratch_types=[
            pltpu.VMEM((ICH, _CR), jnp.int32),
            pltpu.VMEM((N, 3), jnp.float32),
            pltpu.VMEM((MW, 3), jnp.float32),
            pltpu.VMEM((_CR, D), jnp.float32),
            pltpu.VMEM((_CR, D), jnp.float32),
            pltpu.VMEM((MW, 3 * K), jnp.float32),
            pltpu.SemaphoreType.DMA,
            pltpu.SemaphoreType.DMA,
            pltpu.SemaphoreType.DMA,
            pltpu.SemaphoreType.DMA,
        ],
    )


def kernel(input, points, K, next_pts, indices_):
    B, N, D = input.shape
    _, M, Kn = indices_.shape
    idx3 = indices_.astype(jnp.int32).reshape(B, (M * Kn) // _CR, _CR)
    fn = _build(B, N, D, M, Kn)
    fout, pout = fn(input, points, next_pts, idx3)
    return (pout.reshape(B, M, Kn, 3), fout.reshape(B, M, Kn, D),
            next_pts, indices_)


# speedup vs baseline: 4.1625x; 4.1625x over previous
import functools

import jax
import jax.numpy as jnp
from jax import lax
from jax.experimental import pallas as pl
from jax.experimental.pallas import tpu as pltpu
from jax.experimental.pallas import tpu_sc as plsc

_NC = 2
_NS = 16
_NW = _NC * _NS
_CR = 128


@functools.lru_cache(maxsize=None)
def _build(B, N, D, M, K):
    WPB = _NW // B
    MW = M // WPB
    ROWS_W = MW * K
    NCH = ROWS_W // _CR
    ICH = NCH
    MPC = _CR // K

    mesh = plsc.VectorSubcoreMesh(core_axis_name="c", subcore_axis_name="s")

    def body(feat_hbm, pts_hbm, next_hbm, idx_hbm, fout_hbm, pout_hbm,
             idx_ref, ptsv, nextv, fb0, fb1, po, gs0, gs1, ws0, ws1):
        wid = lax.axis_index("s") * _NC + lax.axis_index("c")
        b = wid // WPB
        q = wid % WPB

        pltpu.sync_copy(idx_hbm.at[b, pl.ds(q * ICH, ICH)], idx_ref)
        pltpu.sync_copy(pts_hbm.at[b], ptsv)
        pltpu.sync_copy(next_hbm.at[b, pl.ds(q * MW * 3, MW * 3)], nextv)

        fbufs = (fb0, fb1)
        gsems = (gs0, gs1)
        wsems = (ws0, ws1)
        feat_b = feat_hbm.at[b]
        row0 = q * ROWS_W

        def fire_gather(c, j2):
            pltpu.make_async_copy(
                feat_b.at[idx_ref.at[c]], fbufs[j2], gsems[j2]).start()

        def wait_gather(j2):
            pltpu.make_async_copy(
                feat_b.at[idx_ref.at[0]], fbufs[j2], gsems[j2]).wait()

        def fire_write(c, j2):
            pltpu.make_async_copy(
                fbufs[j2], fout_hbm.at[b, pl.ds(row0 + c * _CR, _CR)],
                wsems[j2]).start()

        def wait_write(j2):
            pltpu.make_async_copy(
                fbufs[j2], fout_hbm.at[b, pl.ds(row0, _CR)], wsems[j2]).wait()

        z16 = jnp.zeros((K,), jnp.int32)
        ki3 = lax.iota(jnp.int32, K) * 3
        zf16 = jnp.zeros((K,), jnp.float32)

        def pts_chunk(c):
            for j in range(MPC):
                m = c * MPC + j
                nb = z16 + m * 3
                iv3 = idx_ref[c, pl.ds(j * K, K)] * 3
                px = plsc.load_gather(ptsv, [iv3])
                py = plsc.load_gather(ptsv, [iv3 + 1])
                pz = plsc.load_gather(ptsv, [iv3 + 2])
                dx = px - plsc.load_gather(nextv, [nb])
                dy = py - plsc.load_gather(nextv, [nb + 1])
                dz = pz - plsc.load_gather(nextv, [nb + 2])
                d2 = dx * dx + dy * dy + dz * dz
                mv = zf16 + jnp.max(d2)
                bits = plsc.bitcast(mv, jnp.int32)
                y = plsc.bitcast(jnp.int32(0x5F3759DF) - (bits >> 1),
                                 jnp.float32)
                half = mv * 0.5
                for _ in range(3):
                    y = y * (1.5 - half * y * y)
                scale = jnp.where(mv == 0.0, jnp.float32(1.0), y)
                ob = ki3 + m * (3 * K)
                plsc.store_scatter(po, [ob], dx * scale)
                plsc.store_scatter(po, [ob + 1], dy * scale)
                plsc.store_scatter(po, [ob + 2], dz * scale)

        fire_gather(0, 0)
        fire_gather(1, 1)

        def pair(cc, carry):
            for j2 in range(2):
                c = cc * 2 + j2
                wait_gather(j2)
                fire_write(c, j2)
                pts_chunk(c)
                wait_write(j2)
                fire_gather(c + 2, j2)
            return carry

        lax.fori_loop(0, NCH // 2 - 1, pair, 0)
        for j2 in range(2):
            c = NCH - 2 + j2
            wait_gather(j2)
            fire_write(c, j2)
            pts_chunk(c)
            wait_write(j2)

        pltpu.sync_copy(po, pout_hbm.at[b, pl.ds(q * MW * 3 * K, MW * 3 * K)])

    return pl.kernel(
        body,
        out_type=[
            jax.ShapeDtypeStruct((B, M * K, D), jnp.float32),
            jax.ShapeDtypeStruct((B, M * 3 * K), jnp.float32),
        ],
        mesh=mesh,
        compiler_params=pltpu.CompilerParams(needs_layout_passes=False,
                                             use_tc_tiling_on_sc=False),
        scratch_types=[
            pltpu.VMEM((ICH, _CR), jnp.int32),
            pltpu.VMEM((N * 3,), jnp.float32),
            pltpu.VMEM((MW * 3,), jnp.float32),
            pltpu.VMEM((_CR, D), jnp.float32),
            pltpu.VMEM((_CR, D), jnp.float32),
            pltpu.VMEM((MW * 3 * K,), jnp.float32),
            pltpu.SemaphoreType.DMA,
            pltpu.SemaphoreType.DMA,
            pltpu.SemaphoreType.DMA,
            pltpu.SemaphoreType.DMA,
        ],
    )


def kernel(input, points, K, next_pts, indices_):
    B, N, D = input.shape
    _, M, Kn = indices_.shape
    idx3 = indices_.astype(jnp.int32).reshape(B, (M * Kn) // _CR, _CR)
    fn = _build(B, N, D, M, Kn)
    fout, pout = fn(input, points.reshape(B, N * 3),
                    next_pts.reshape(B, M * 3), idx3)
    return (pout.reshape(B, M, Kn, 3), fout.reshape(B, M, Kn, D),
            next_pts, indices_)


# speedup vs baseline: 5.6283x; 1.3521x over previous
import functools

import jax
import jax.numpy as jnp
from jax import lax
from jax.experimental import pallas as pl
from jax.experimental.pallas import tpu as pltpu
from jax.experimental.pallas import tpu_sc as plsc

_NC = 2
_NS = 16
_NW = _NC * _NS
_CR = 128


@functools.lru_cache(maxsize=None)
def _build(B, N, D, M, K):
    WPB = _NW // B
    MW = M // WPB
    ROWS_W = MW * K
    NCH = ROWS_W // _CR
    ICH = NCH
    MPC = _CR // K

    mesh = plsc.VectorSubcoreMesh(core_axis_name="c", subcore_axis_name="s")

    def body(feat_hbm, pts_hbm, next_hbm, idx_hbm, fout_hbm, pout_hbm,
             idx_ref, ptsv, nextv, fb0, fb1, pov, gs0, gs1, ws0, ws1):
        wid = lax.axis_index("s") * _NC + lax.axis_index("c")
        b = wid // WPB
        q = wid % WPB

        pltpu.sync_copy(idx_hbm.at[b, pl.ds(q * ICH, ICH)], idx_ref)
        pltpu.sync_copy(pts_hbm.at[b], ptsv)
        pltpu.sync_copy(next_hbm.at[b, pl.ds(q * MW * 3, MW * 3)], nextv)

        fbufs = (fb0, fb1)
        gsems = (gs0, gs1)
        wsems = (ws0, ws1)
        feat_b = feat_hbm.at[b]
        m0w = q * MW

        def fire_gather(c, j2):
            pltpu.make_async_copy(
                feat_b.at[idx_ref.at[c]], fbufs[j2], gsems[j2]).start()

        def wait_gather(j2):
            pltpu.make_async_copy(
                feat_b.at[idx_ref.at[0]], fbufs[j2], gsems[j2]).wait()

        def fire_write(c, j2):
            for j in range(MPC):
                pltpu.make_async_copy(
                    fbufs[j2].at[pl.ds(j * K, K)],
                    fout_hbm.at[b, m0w + c * MPC + j, :, :],
                    wsems[j2]).start()

        def wait_write(j2):
            pltpu.make_async_copy(
                feat_b.at[pl.ds(0, _CR)], fbufs[j2], wsems[j2]).wait()

        z16 = jnp.zeros((K,), jnp.int32)
        ki3 = lax.iota(jnp.int32, K) * 3
        zf16 = jnp.zeros((K,), jnp.float32)

        def pts_chunk(c):
            for j in range(MPC):
                m = c * MPC + j
                m16 = z16 + m
                nb = z16 + m * 3
                iv3 = idx_ref[c, pl.ds(j * K, K)] * 3
                px = plsc.load_gather(ptsv, [iv3])
                py = plsc.load_gather(ptsv, [iv3 + 1])
                pz = plsc.load_gather(ptsv, [iv3 + 2])
                dx = px - plsc.load_gather(nextv, [nb])
                dy = py - plsc.load_gather(nextv, [nb + 1])
                dz = pz - plsc.load_gather(nextv, [nb + 2])
                d2 = dx * dx + dy * dy + dz * dz
                mv = zf16 + jnp.max(d2)
                bits = plsc.bitcast(mv, jnp.int32)
                y = plsc.bitcast(jnp.int32(0x5F3759DF) - (bits >> 1),
                                 jnp.float32)
                half = mv * 0.5
                for _ in range(3):
                    y = y * (1.5 - half * y * y)
                scale = jnp.where(mv == 0.0, jnp.float32(1.0), y)
                plsc.store_scatter(pov, [m16, ki3], dx * scale)
                plsc.store_scatter(pov, [m16, ki3 + 1], dy * scale)
                plsc.store_scatter(pov, [m16, ki3 + 2], dz * scale)

        fire_gather(0, 0)
        fire_gather(1, 1)

        def pair(cc, carry):
            for j2 in range(2):
                c = cc * 2 + j2
                wait_gather(j2)
                fire_write(c, j2)
                pts_chunk(c)
                wait_write(j2)
                fire_gather(c + 2, j2)
            return carry

        lax.fori_loop(0, NCH // 2 - 1, pair, 0)
        for j2 in range(2):
            c = NCH - 2 + j2
            wait_gather(j2)
            fire_write(c, j2)
            pts_chunk(c)
            wait_write(j2)

        pltpu.sync_copy(pov, pout_hbm.at[b, pl.ds(q * MW, MW), :])

    return pl.kernel(
        body,
        out_type=[
            jax.ShapeDtypeStruct((B, M, K, D), jnp.float32),
            jax.ShapeDtypeStruct((B, M, 3 * K), jnp.float32),
        ],
        mesh=mesh,
        compiler_params=pltpu.CompilerParams(needs_layout_passes=False,
                                             use_tc_tiling_on_sc=False),
        scratch_types=[
            pltpu.VMEM((ICH, _CR), jnp.int32),
            pltpu.VMEM((N * 3,), jnp.float32),
            pltpu.VMEM((MW * 3,), jnp.float32),
            pltpu.VMEM((_CR, D), jnp.float32),
            pltpu.VMEM((_CR, D), jnp.float32),
            pltpu.VMEM((MW, 3 * K), jnp.float32),
            pltpu.SemaphoreType.DMA,
            pltpu.SemaphoreType.DMA,
            pltpu.SemaphoreType.DMA,
            pltpu.SemaphoreType.DMA,
        ],
    )


def kernel(input, points, K, next_pts, indices_):
    B, N, D = input.shape
    _, M, Kn = indices_.shape
    idx3 = indices_.astype(jnp.int32).reshape(B, (M * Kn) // _CR, _CR)
    fn = _build(B, N, D, M, Kn)
    fout, pout = fn(input, points.reshape(B, N * 3),
                    next_pts.reshape(B, M * 3), idx3)
    return (pout.reshape(B, M, Kn, 3), fout, next_pts, indices_)
